# baseline (speedup 1.0000x reference)
import functools

import jax
import jax.numpy as jnp
from jax import lax
from jax.experimental import pallas as pl
from jax.experimental.pallas import tpu as pltpu
from jax.experimental.pallas import tpu_sc as plsc

F32 = jnp.float32
I32 = jnp.int32

L = 2
S = 2048
D = 1024
DIN = 1024
FF = 4096
E = 8
K = 2
CAP = (K * S) // E
ECAP = E * CAP
SLOTS_PAD = ECAP + 16
FC = 2048
NFC = FF // FC
EG = 2
NG = E // EG
GSLOT = EG * CAP
TB = 256
CS = 256

_NC = 2
_NS = 16
_NW = _NC * _NS


def _ln(t, g, b):
    m = jnp.mean(t, axis=-1, keepdims=True)
    v = jnp.mean((t - m) ** 2, axis=-1, keepdims=True)
    return (t - m) / jnp.sqrt(v + 1e-5) * g + b


def _in_body(x_ref, w_ref, b_ref, g_ref, bb_ref, o_ref):
    t = jnp.dot(x_ref[...], w_ref[...], preferred_element_type=F32) + b_ref[...]
    o_ref[...] = _ln(t, g_ref[...], bb_ref[...])


def _k_in(x, W_in, b_in, g, b):
    return pl.pallas_call(
        _in_body,
        grid=(S // TB,),
        in_specs=[
            pl.BlockSpec((TB, DIN), lambda i: (i, 0)),
            pl.BlockSpec((DIN, D), lambda i: (0, 0)),
            pl.BlockSpec((1, D), lambda i: (0, 0)),
            pl.BlockSpec((1, D), lambda i: (0, 0)),
            pl.BlockSpec((1, D), lambda i: (0, 0)),
        ],
        out_specs=pl.BlockSpec((TB, D), lambda i: (i, 0)),
        out_shape=jax.ShapeDtypeStruct((S, D), F32),
    )(x, W_in, b_in.reshape(1, D), g.reshape(1, D), b.reshape(1, D))


def _cumsum0(m, tril):
    chunks = []
    carry = jnp.zeros((1, E), F32)
    for c in range(S // CS):
        blk = m[c * CS:(c + 1) * CS, :]
        cum = jnp.dot(tril, blk, preferred_element_type=F32) + carry
        chunks.append(cum)
        carry = cum[CS - 1:CS, :]
    return jnp.concatenate(chunks, axis=0)


def _route_body(h_ref, g_ref, b_ref, wg_ref, bg_ref,
                ssc0_ref, ssc1_ref, sg0_ref, sg1_ref, g0_ref, g1_ref):
    z = _ln(h_ref[...], g_ref[...], b_ref[...])
    logits = jnp.dot(z, wg_ref[...], preferred_element_type=F32) + bg_ref[...]
    p = jax.nn.softmax(logits, axis=-1)
    ie = lax.broadcasted_iota(I32, (S, E), 1)
    v0 = jnp.max(p, axis=-1, keepdims=True)
    e0 = jnp.min(jnp.where(p == v0, ie, E), axis=-1, keepdims=True)
    p1 = jnp.where(ie == e0, -jnp.inf, p)
    v1 = jnp.max(p1, axis=-1, keepdims=True)
    e1 = jnp.min(jnp.where(p1 == v1, ie, E), axis=-1, keepdims=True)
    den = v0 + v1 + 1e-9
    m0 = (ie == e0).astype(F32)
    m1 = (ie == e1).astype(F32)
    ir = lax.broadcasted_iota(I32, (CS, CS), 0)
    ic = lax.broadcasted_iota(I32, (CS, CS), 1)
    tril = (ir >= ic).astype(F32)
    c0 = _cumsum0(m0, tril)
    c1 = _cumsum0(m1, tril)
    total0 = c0[S - 1:S, :]
    pos0 = jnp.sum(jnp.where(ie == e0, c0 - 1.0, 0.0), axis=-1, keepdims=True)
    pos1 = jnp.sum(jnp.where(ie == e1, c1 - 1.0 + total0, 0.0), axis=-1,
                   keepdims=True)
    pos0 = pos0.astype(I32)
    pos1 = pos1.astype(I32)
    keep0 = pos0 < CAP
    keep1 = pos1 < CAP
    slot0 = e0 * CAP + pos0
    slot1 = e1 * CAP + pos1
    ssc0_ref[...] = jnp.where(keep0, slot0, ECAP)
    ssc1_ref[...] = jnp.where(keep1, slot1, ECAP)
    sg0_ref[...] = jnp.where(keep0, slot0, 0)
    sg1_ref[...] = jnp.where(keep1, slot1, 0)
    g0_ref[...] = v0 / den * keep0.astype(F32)
    g1_ref[...] = v1 / den * keep1.astype(F32)


def _k_route(h, g, b, Wg_l, bg_l):
    col_i = jax.ShapeDtypeStruct((S, 1), I32)
    col_f = jax.ShapeDtypeStruct((S, 1), F32)
    return pl.pallas_call(
        _route_body,
        out_shape=(col_i, col_i, col_i, col_i, col_f, col_f),
    )(h, g.reshape(1, D), b.reshape(1, D), Wg_l, bg_l.reshape(1, E))


def _ffn_body(x_ref, g_ref, b_ref, w1_ref, b1_ref, w2_ref, b2_ref, o_ref):
    fc = pl.program_id(1)

    @pl.when(fc == 0)
    def _():
        o_ref[...] = jnp.broadcast_to(b2_ref[0], (CAP, D))

    z = _ln(x_ref[...], g_ref[...], b_ref[...])
    h = jnp.maximum(
        jnp.dot(z, w1_ref[0, 0], preferred_element_type=F32)
        + b1_ref[0], 0.0)
    o_ref[...] += jnp.dot(h, w2_ref[0, 0], preferred_element_type=F32)


def _k_ffn(xin, lng, lnb, W1, b1, W2, b2, l):
    return pl.pallas_call(
        _ffn_body,
        grid=(E, NFC),
        in_specs=[
            pl.BlockSpec((CAP, D), lambda e, f: (e, 0)),
            pl.BlockSpec((1, D), lambda e, f: (0, 0)),
            pl.BlockSpec((1, D), lambda e, f: (0, 0)),
            pl.BlockSpec((1, 1, D, FC), lambda e, f, l=l: (l, e, 0, f)),
            pl.BlockSpec((1, 1, FC), lambda e, f, l=l: (l * E + e, 0, f)),
            pl.BlockSpec((1, 1, FC, D), lambda e, f, l=l: (l, e, f, 0)),
            pl.BlockSpec((1, 1, D), lambda e, f, l=l: (l * E + e, 0, 0)),
        ],
        out_specs=pl.BlockSpec((CAP, D), lambda e, f: (e, 0)),
        out_shape=jax.ShapeDtypeStruct((ECAP, D), F32),
        compiler_params=pltpu.CompilerParams(
            dimension_semantics=("arbitrary", "arbitrary")),
    )(xin, lng.reshape(1, D), lnb.reshape(1, D),
      W1, b1.reshape(L * E, 1, FF), W2, b2.reshape(L * E, 1, D))


def _cmb_body(r_ref, c0_ref, c1_ref, g0_ref, g1_ref, o_ref):
    o_ref[...] = (r_ref[...] + g0_ref[...] * c0_ref[...]
                  + g1_ref[...] * c1_ref[...])


def _k_cmb(res, c0, c1, g0, g1):
    return pl.pallas_call(
        _cmb_body,
        grid=(S // TB,),
        in_specs=[
            pl.BlockSpec((TB, D), lambda i: (i, 0)),
            pl.BlockSpec((TB, D), lambda i: (i, 0)),
            pl.BlockSpec((TB, D), lambda i: (i, 0)),
            pl.BlockSpec((TB, 1), lambda i: (i, 0)),
            pl.BlockSpec((TB, 1), lambda i: (i, 0)),
        ],
        out_specs=pl.BlockSpec((TB, D), lambda i: (i, 0)),
        out_shape=jax.ShapeDtypeStruct((S, D), F32),
    )(res, c0, c1, g0, g1)


def _out_body(r_ref, c0_ref, c1_ref, g0_ref, g1_ref, w_ref, b_ref,
              g_ref, bb_ref, o_ref):
    y = (r_ref[...] + g0_ref[...] * c0_ref[...] + g1_ref[...] * c1_ref[...])
    t = jnp.dot(y, w_ref[...], preferred_element_type=F32) + b_ref[...]
    o_ref[...] = _ln(t, g_ref[...], bb_ref[...])


def _k_out(res, c0, c1, g0, g1, W_out, b_out, g, b):
    return pl.pallas_call(
        _out_body,
        grid=(S // TB,),
        in_specs=[
            pl.BlockSpec((TB, D), lambda i: (i, 0)),
            pl.BlockSpec((TB, D), lambda i: (i, 0)),
            pl.BlockSpec((TB, D), lambda i: (i, 0)),
            pl.BlockSpec((TB, 1), lambda i: (i, 0)),
            pl.BlockSpec((TB, 1), lambda i: (i, 0)),
            pl.BlockSpec((D, DIN), lambda i: (0, 0)),
            pl.BlockSpec((1, DIN), lambda i: (0, 0)),
            pl.BlockSpec((1, DIN), lambda i: (0, 0)),
            pl.BlockSpec((1, DIN), lambda i: (0, 0)),
        ],
        out_specs=pl.BlockSpec((TB, DIN), lambda i: (i, 0)),
        out_shape=jax.ShapeDtypeStruct((S, DIN), F32),
    )(res, c0, c1, g0, g1, W_out, b_out.reshape(1, DIN),
      g.reshape(1, DIN), b.reshape(1, DIN))


def _slotmap_body(s0_hbm, s1_hbm, tfs_hbm, tfs_v, s0_v, s1_v):
    wid = lax.axis_index("s") * _NC + lax.axis_index("c")

    @pl.when(wid == 0)
    def _():
        def init(j, _):
            tfs_v[pl.ds(j * 16, 16)] = jnp.zeros((16,), I32)
            return 0
        lax.fori_loop(0, SLOTS_PAD // 16, init, 0)
        pltpu.sync_copy(s0_hbm, s0_v)
        pltpu.sync_copy(s1_hbm, s1_v)

        def scat(j, _):
            toks = lax.iota(I32, 16) + j * 16
            plsc.store_scatter(tfs_v, [s0_v[pl.ds(j * 16, 16)]], toks)
            plsc.store_scatter(tfs_v, [s1_v[pl.ds(j * 16, 16)]], toks)
            return 0
        lax.fori_loop(0, S // 16, scat, 0)
        pltpu.sync_copy(tfs_v, tfs_hbm)


@functools.lru_cache(maxsize=None)
def _k_slotmap():
    return pl.kernel(
        _slotmap_body,
        mesh=plsc.VectorSubcoreMesh(core_axis_name="c", subcore_axis_name="s"),
        compiler_params=pltpu.CompilerParams(needs_layout_passes=False),
        out_type=jax.ShapeDtypeStruct((SLOTS_PAD,), I32),
        scratch_types=[
            pltpu.VMEM((SLOTS_PAD,), I32),
            pltpu.VMEM((S,), I32),
            pltpu.VMEM((S,), I32),
        ],
    )


def _pipelined_gather(src_hbm, chunks, bufs, gs, ws):
    n = len(chunks)
    g = [None] * n
    w = [None] * n
    for c in range(n):
        b = c % 2
        if c >= 2:
            w[c - 2].wait()
        g[c] = pltpu.async_copy(src_hbm.at[chunks[c][0]], bufs.at[b], gs[b])
        if c >= 1:
            g[c - 1].wait()
            w[c - 1] = pltpu.async_copy(bufs.at[(c - 1) % 2],
                                        chunks[c - 1][1], ws[(c - 1) % 2])
    g[n - 1].wait()
    w[n - 1] = pltpu.async_copy(bufs.at[(n - 1) % 2], chunks[n - 1][1],
                                ws[(n - 1) % 2])
    if n >= 2:
        w[n - 2].wait()
    w[n - 1].wait()


def _dispatch_body(z_hbm, tfs_hbm, out_hbm, idx_v, bufs, gs0, gs1, ws0, ws1):
    wid = lax.axis_index("s") * _NC + lax.axis_index("c")
    base = wid * (ECAP // _NW)
    pltpu.sync_copy(tfs_hbm.at[pl.ds(base, 128)], idx_v)
    chunks = [(idx_v.at[pl.ds(c * 32, 32)],
               out_hbm.at[pl.ds(base + c * 32, 32)]) for c in range(4)]
    _pipelined_gather(z_hbm, chunks, bufs, (gs0, gs1), (ws0, ws1))


@functools.lru_cache(maxsize=None)
def _k_dispatch():
    return pl.kernel(
        _dispatch_body,
        mesh=plsc.VectorSubcoreMesh(core_axis_name="c", subcore_axis_name="s"),
        compiler_params=pltpu.CompilerParams(needs_layout_passes=False),
        out_type=jax.ShapeDtypeStruct((ECAP, D), F32),
        scratch_types=[
            pltpu.VMEM((128,), I32),
            pltpu.VMEM((2, 32, D), F32),
            pltpu.SemaphoreType.DMA,
            pltpu.SemaphoreType.DMA,
            pltpu.SemaphoreType.DMA,
            pltpu.SemaphoreType.DMA,
        ],
    )


def _gatherout_body(ffn_hbm, sg0_hbm, sg1_hbm, c0_hbm, c1_hbm,
                    i0_v, i1_v, bufs, gs0, gs1, ws0, ws1):
    wid = lax.axis_index("s") * _NC + lax.axis_index("c")
    base = wid * (S // _NW)
    pltpu.sync_copy(sg0_hbm.at[pl.ds(base, 64)], i0_v)
    pltpu.sync_copy(sg1_hbm.at[pl.ds(base, 64)], i1_v)
    chunks = []
    for c in range(2):
        chunks.append((i0_v.at[pl.ds(c * 32, 32)],
                       c0_hbm.at[pl.ds(base + c * 32, 32)]))
        chunks.append((i1_v.at[pl.ds(c * 32, 32)],
                       c1_hbm.at[pl.ds(base + c * 32, 32)]))
    _pipelined_gather(ffn_hbm, chunks, bufs, (gs0, gs1), (ws0, ws1))


@functools.lru_cache(maxsize=None)
def _k_gatherout():
    return pl.kernel(
        _gatherout_body,
        mesh=plsc.VectorSubcoreMesh(core_axis_name="c", subcore_axis_name="s"),
        compiler_params=pltpu.CompilerParams(needs_layout_passes=False),
        out_type=(jax.ShapeDtypeStruct((S, D), F32),
                  jax.ShapeDtypeStruct((S, D), F32)),
        scratch_types=[
            pltpu.VMEM((64,), I32),
            pltpu.VMEM((64,), I32),
            pltpu.VMEM((2, 32, D), F32),
            pltpu.SemaphoreType.DMA,
            pltpu.SemaphoreType.DMA,
            pltpu.SemaphoreType.DMA,
            pltpu.SemaphoreType.DMA,
        ],
    )


def _sc_slotmap(s0, s1):
    return _k_slotmap()(s0, s1)


def _sc_dispatch(z, tfs):
    return _k_dispatch()(z, tfs)


def _sc_gatherout(ffn, sg0, sg1):
    return _k_gatherout()(ffn, sg0, sg1)


def _moe_layer(h, g, b, Wg_l, bg_l, W1, b1, W2, b2, l):
    ssc0, ssc1, sg0, sg1, g0, g1 = _k_route(h, g, b, Wg_l, bg_l)
    tfs = _sc_slotmap(ssc0.reshape(S), ssc1.reshape(S))
    xin = _sc_dispatch(h, tfs)
    ffn = _k_ffn(xin, g, b, W1, b1, W2, b2, l)
    c0, c1 = _sc_gatherout(ffn, sg0.reshape(S), sg1.reshape(S))
    return c0, c1, g0, g1


def kernel(x, W_in, b_in, ln_in_g, ln_in_b, ln_g, ln_b, Wg, bg,
           W1, b1, W2, b2, W_out, b_out, ln_out_g, ln_out_b):
    x2 = x.reshape(S, DIN)
    h = _k_in(x2, W_in, b_in, ln_in_g, ln_in_b)
    c0, c1, g0, g1 = _moe_layer(h, ln_g[0], ln_b[0], Wg[0], bg[0],
                                W1, b1, W2, b2, 0)
    h = _k_cmb(h, c0, c1, g0, g1)
    c0, c1, g0, g1 = _moe_layer(h, ln_g[1], ln_b[1], Wg[1], bg[1],
                                W1, b1, W2, b2, 1)
    out = _k_out(h, c0, c1, g0, g1, W_out, b_out, ln_out_g, ln_out_b)
    return out.reshape(1, S, DIN)


# speedup vs baseline: 1.0319x; 1.0319x over previous
import functools

import jax
import jax.numpy as jnp
from jax import lax
from jax.experimental import pallas as pl
from jax.experimental.pallas import tpu as pltpu
from jax.experimental.pallas import tpu_sc as plsc

F32 = jnp.float32
I32 = jnp.int32

L = 2
S = 2048
D = 1024
DIN = 1024
FF = 4096
E = 8
K = 2
CAP = (K * S) // E
ECAP = E * CAP
SLOTS_PAD = ECAP + 16
FC = 2048
NFC = FF // FC
EG = 2
NG = E // EG
GSLOT = EG * CAP
TB = 256
CS = 256

_NC = 2
_NS = 16
_NW = _NC * _NS


def _ln(t, g, b):
    m = jnp.mean(t, axis=-1, keepdims=True)
    v = jnp.mean((t - m) ** 2, axis=-1, keepdims=True)
    return (t - m) / jnp.sqrt(v + 1e-5) * g + b


def _in_body(x_ref, w_ref, b_ref, g_ref, bb_ref, o_ref):
    t = jnp.dot(x_ref[...], w_ref[...], preferred_element_type=F32) + b_ref[...]
    o_ref[...] = _ln(t, g_ref[...], bb_ref[...])


def _k_in(x, W_in, b_in, g, b):
    return pl.pallas_call(
        _in_body,
        grid=(S // TB,),
        in_specs=[
            pl.BlockSpec((TB, DIN), lambda i: (i, 0)),
            pl.BlockSpec((DIN, D), lambda i: (0, 0)),
            pl.BlockSpec((1, D), lambda i: (0, 0)),
            pl.BlockSpec((1, D), lambda i: (0, 0)),
            pl.BlockSpec((1, D), lambda i: (0, 0)),
        ],
        out_specs=pl.BlockSpec((TB, D), lambda i: (i, 0)),
        out_shape=jax.ShapeDtypeStruct((S, D), F32),
    )(x, W_in, b_in.reshape(1, D), g.reshape(1, D), b.reshape(1, D))


def _cumsum0(m, tril):
    chunks = []
    carry = jnp.zeros((1, E), F32)
    for c in range(S // CS):
        blk = m[c * CS:(c + 1) * CS, :]
        cum = jnp.dot(tril, blk, preferred_element_type=F32) + carry
        chunks.append(cum)
        carry = cum[CS - 1:CS, :]
    return jnp.concatenate(chunks, axis=0)


def _route_body(h_ref, g_ref, b_ref, wg_ref, bg_ref,
                z_ref, ssc0_ref, ssc1_ref, sg0_ref, sg1_ref, g0_ref, g1_ref):
    z = _ln(h_ref[...], g_ref[...], b_ref[...])
    z_ref[...] = z
    logits = jnp.dot(z, wg_ref[...], preferred_element_type=F32) + bg_ref[...]
    p = jax.nn.softmax(logits, axis=-1)
    ie = lax.broadcasted_iota(I32, (S, E), 1)
    v0 = jnp.max(p, axis=-1, keepdims=True)
    e0 = jnp.min(jnp.where(p == v0, ie, E), axis=-1, keepdims=True)
    p1 = jnp.where(ie == e0, -jnp.inf, p)
    v1 = jnp.max(p1, axis=-1, keepdims=True)
    e1 = jnp.min(jnp.where(p1 == v1, ie, E), axis=-1, keepdims=True)
    den = v0 + v1 + 1e-9
    m0 = (ie == e0).astype(F32)
    m1 = (ie == e1).astype(F32)
    ir = lax.broadcasted_iota(I32, (CS, CS), 0)
    ic = lax.broadcasted_iota(I32, (CS, CS), 1)
    tril = (ir >= ic).astype(F32)
    c0 = _cumsum0(m0, tril)
    c1 = _cumsum0(m1, tril)
    total0 = c0[S - 1:S, :]
    pos0 = jnp.sum(jnp.where(ie == e0, c0 - 1.0, 0.0), axis=-1, keepdims=True)
    pos1 = jnp.sum(jnp.where(ie == e1, c1 - 1.0 + total0, 0.0), axis=-1,
                   keepdims=True)
    pos0 = pos0.astype(I32)
    pos1 = pos1.astype(I32)
    keep0 = pos0 < CAP
    keep1 = pos1 < CAP
    slot0 = e0 * CAP + pos0
    slot1 = e1 * CAP + pos1
    ssc0_ref[...] = jnp.where(keep0, slot0, ECAP)
    ssc1_ref[...] = jnp.where(keep1, slot1, ECAP)
    sg0_ref[...] = jnp.where(keep0, slot0, 0)
    sg1_ref[...] = jnp.where(keep1, slot1, 0)
    g0_ref[...] = v0 / den * keep0.astype(F32)
    g1_ref[...] = v1 / den * keep1.astype(F32)


def _k_route(h, g, b, Wg_l, bg_l):
    col_i = jax.ShapeDtypeStruct((S, 1), I32)
    col_f = jax.ShapeDtypeStruct((S, 1), F32)
    return pl.pallas_call(
        _route_body,
        out_shape=(jax.ShapeDtypeStruct((S, D), F32),
                   col_i, col_i, col_i, col_i, col_f, col_f),
    )(h, g.reshape(1, D), b.reshape(1, D), Wg_l, bg_l.reshape(1, E))


def _ffn_body(x_ref, w1_ref, b1_ref, w2_ref, b2_ref, o_ref):
    fc = pl.program_id(1)

    @pl.when(fc == 0)
    def _():
        o_ref[...] = jnp.broadcast_to(b2_ref[0], (CAP, D))

    h = jnp.maximum(
        jnp.dot(x_ref[...], w1_ref[0, 0], preferred_element_type=F32)
        + b1_ref[0], 0.0)
    o_ref[...] += jnp.dot(h, w2_ref[0, 0], preferred_element_type=F32)


def _k_ffn(xin, W1, b1, W2, b2, l):
    return pl.pallas_call(
        _ffn_body,
        grid=(E, NFC),
        in_specs=[
            pl.BlockSpec((CAP, D), lambda e, f: (e, 0)),
            pl.BlockSpec((1, 1, D, FC), lambda e, f, l=l: (l, e, 0, f)),
            pl.BlockSpec((1, 1, FC), lambda e, f, l=l: (l * E + e, 0, f)),
            pl.BlockSpec((1, 1, FC, D), lambda e, f, l=l: (l, e, f, 0)),
            pl.BlockSpec((1, 1, D), lambda e, f, l=l: (l * E + e, 0, 0)),
        ],
        out_specs=pl.BlockSpec((CAP, D), lambda e, f: (e, 0)),
        out_shape=jax.ShapeDtypeStruct((ECAP, D), F32),
        compiler_params=pltpu.CompilerParams(
            dimension_semantics=("arbitrary", "arbitrary")),
    )(xin, W1, b1.reshape(L * E, 1, FF), W2, b2.reshape(L * E, 1, D))


def _cmb_body(r_ref, c0_ref, c1_ref, g0_ref, g1_ref, o_ref):
    o_ref[...] = (r_ref[...] + g0_ref[...] * c0_ref[...]
                  + g1_ref[...] * c1_ref[...])


def _k_cmb(res, c0, c1, g0, g1):
    return pl.pallas_call(
        _cmb_body,
        grid=(S // TB,),
        in_specs=[
            pl.BlockSpec((TB, D), lambda i: (i, 0)),
            pl.BlockSpec((TB, D), lambda i: (i, 0)),
            pl.BlockSpec((TB, D), lambda i: (i, 0)),
            pl.BlockSpec((TB, 1), lambda i: (i, 0)),
            pl.BlockSpec((TB, 1), lambda i: (i, 0)),
        ],
        out_specs=pl.BlockSpec((TB, D), lambda i: (i, 0)),
        out_shape=jax.ShapeDtypeStruct((S, D), F32),
    )(res, c0, c1, g0, g1)


def _out_body(r_ref, c0_ref, c1_ref, g0_ref, g1_ref, w_ref, b_ref,
              g_ref, bb_ref, o_ref):
    y = (r_ref[...] + g0_ref[...] * c0_ref[...] + g1_ref[...] * c1_ref[...])
    t = jnp.dot(y, w_ref[...], preferred_element_type=F32) + b_ref[...]
    o_ref[...] = _ln(t, g_ref[...], bb_ref[...])


def _k_out(res, c0, c1, g0, g1, W_out, b_out, g, b):
    return pl.pallas_call(
        _out_body,
        grid=(S // TB,),
        in_specs=[
            pl.BlockSpec((TB, D), lambda i: (i, 0)),
            pl.BlockSpec((TB, D), lambda i: (i, 0)),
            pl.BlockSpec((TB, D), lambda i: (i, 0)),
            pl.BlockSpec((TB, 1), lambda i: (i, 0)),
            pl.BlockSpec((TB, 1), lambda i: (i, 0)),
            pl.BlockSpec((D, DIN), lambda i: (0, 0)),
            pl.BlockSpec((1, DIN), lambda i: (0, 0)),
            pl.BlockSpec((1, DIN), lambda i: (0, 0)),
            pl.BlockSpec((1, DIN), lambda i: (0, 0)),
        ],
        out_specs=pl.BlockSpec((TB, DIN), lambda i: (i, 0)),
        out_shape=jax.ShapeDtypeStruct((S, DIN), F32),
    )(res, c0, c1, g0, g1, W_out, b_out.reshape(1, DIN),
      g.reshape(1, DIN), b.reshape(1, DIN))


def _slotmap_body(s0_hbm, s1_hbm, tfs_hbm, tfs_v, s0_v, s1_v):
    wid = lax.axis_index("s") * _NC + lax.axis_index("c")

    @pl.when(wid == 0)
    def _():
        def init(j, _):
            tfs_v[pl.ds(j * 16, 16)] = jnp.zeros((16,), I32)
            return 0
        lax.fori_loop(0, SLOTS_PAD // 16, init, 0)
        pltpu.sync_copy(s0_hbm, s0_v)
        pltpu.sync_copy(s1_hbm, s1_v)

        def scat(j, _):
            toks = lax.iota(I32, 16) + j * 16
            plsc.store_scatter(tfs_v, [s0_v[pl.ds(j * 16, 16)]], toks)
            plsc.store_scatter(tfs_v, [s1_v[pl.ds(j * 16, 16)]], toks)
            return 0
        lax.fori_loop(0, S // 16, scat, 0)
        pltpu.sync_copy(tfs_v, tfs_hbm)


@functools.lru_cache(maxsize=None)
def _k_slotmap():
    return pl.kernel(
        _slotmap_body,
        mesh=plsc.VectorSubcoreMesh(core_axis_name="c", subcore_axis_name="s"),
        compiler_params=pltpu.CompilerParams(needs_layout_passes=False),
        out_type=jax.ShapeDtypeStruct((SLOTS_PAD,), I32),
        scratch_types=[
            pltpu.VMEM((SLOTS_PAD,), I32),
            pltpu.VMEM((S,), I32),
            pltpu.VMEM((S,), I32),
        ],
    )


def _pipelined_gather(src_hbm, chunks, bufs, gs, ws):
    n = len(chunks)
    g = [None] * n
    w = [None] * n
    for c in range(n):
        b = c % 2
        if c >= 2:
            w[c - 2].wait()
        g[c] = pltpu.async_copy(src_hbm.at[chunks[c][0]], bufs.at[b], gs[b])
        if c >= 1:
            g[c - 1].wait()
            w[c - 1] = pltpu.async_copy(bufs.at[(c - 1) % 2],
                                        chunks[c - 1][1], ws[(c - 1) % 2])
    g[n - 1].wait()
    w[n - 1] = pltpu.async_copy(bufs.at[(n - 1) % 2], chunks[n - 1][1],
                                ws[(n - 1) % 2])
    if n >= 2:
        w[n - 2].wait()
    w[n - 1].wait()


def _dispatch_body(z_hbm, tfs_hbm, out_hbm, idx_v, bufs, gs0, gs1, ws0, ws1):
    wid = lax.axis_index("s") * _NC + lax.axis_index("c")
    base = wid * (ECAP // _NW)
    pltpu.sync_copy(tfs_hbm.at[pl.ds(base, 128)], idx_v)
    chunks = [(idx_v.at[pl.ds(c * 32, 32)],
               out_hbm.at[pl.ds(base + c * 32, 32)]) for c in range(4)]
    _pipelined_gather(z_hbm, chunks, bufs, (gs0, gs1), (ws0, ws1))


@functools.lru_cache(maxsize=None)
def _k_dispatch():
    return pl.kernel(
        _dispatch_body,
        mesh=plsc.VectorSubcoreMesh(core_axis_name="c", subcore_axis_name="s"),
        compiler_params=pltpu.CompilerParams(needs_layout_passes=False),
        out_type=jax.ShapeDtypeStruct((ECAP, D), F32),
        scratch_types=[
            pltpu.VMEM((128,), I32),
            pltpu.VMEM((2, 32, D), F32),
            pltpu.SemaphoreType.DMA,
            pltpu.SemaphoreType.DMA,
            pltpu.SemaphoreType.DMA,
            pltpu.SemaphoreType.DMA,
        ],
    )


def _gatherout_body(ffn_hbm, sg0_hbm, sg1_hbm, c0_hbm, c1_hbm,
                    i0_v, i1_v, bufs, gs0, gs1, ws0, ws1):
    wid = lax.axis_index("s") * _NC + lax.axis_index("c")
    base = wid * (S // _NW)
    pltpu.sync_copy(sg0_hbm.at[pl.ds(base, 64)], i0_v)
    pltpu.sync_copy(sg1_hbm.at[pl.ds(base, 64)], i1_v)
    chunks = []
    for c in range(2):
        chunks.append((i0_v.at[pl.ds(c * 32, 32)],
                       c0_hbm.at[pl.ds(base + c * 32, 32)]))
        chunks.append((i1_v.at[pl.ds(c * 32, 32)],
                       c1_hbm.at[pl.ds(base + c * 32, 32)]))
    _pipelined_gather(ffn_hbm, chunks, bufs, (gs0, gs1), (ws0, ws1))


@functools.lru_cache(maxsize=None)
def _k_gatherout():
    return pl.kernel(
        _gatherout_body,
        mesh=plsc.VectorSubcoreMesh(core_axis_name="c", subcore_axis_name="s"),
        compiler_params=pltpu.CompilerParams(needs_layout_passes=False),
        out_type=(jax.ShapeDtypeStruct((S, D), F32),
                  jax.ShapeDtypeStruct((S, D), F32)),
        scratch_types=[
            pltpu.VMEM((64,), I32),
            pltpu.VMEM((64,), I32),
            pltpu.VMEM((2, 32, D), F32),
            pltpu.SemaphoreType.DMA,
            pltpu.SemaphoreType.DMA,
            pltpu.SemaphoreType.DMA,
            pltpu.SemaphoreType.DMA,
        ],
    )


def _sc_slotmap(s0, s1):
    return _k_slotmap()(s0, s1)


def _sc_dispatch(z, tfs):
    return _k_dispatch()(z, tfs)


def _sc_gatherout(ffn, sg0, sg1):
    return _k_gatherout()(ffn, sg0, sg1)


def _moe_layer(h, g, b, Wg_l, bg_l, W1, b1, W2, b2, l):
    z, ssc0, ssc1, sg0, sg1, g0, g1 = _k_route(h, g, b, Wg_l, bg_l)
    tfs = _sc_slotmap(ssc0.reshape(S), ssc1.reshape(S))
    xin = _sc_dispatch(z, tfs)
    ffn = _k_ffn(xin, W1, b1, W2, b2, l)
    c0, c1 = _sc_gatherout(ffn, sg0.reshape(S), sg1.reshape(S))
    return c0, c1, g0, g1


def kernel(x, W_in, b_in, ln_in_g, ln_in_b, ln_g, ln_b, Wg, bg,
           W1, b1, W2, b2, W_out, b_out, ln_out_g, ln_out_b):
    x2 = x.reshape(S, DIN)
    h = _k_in(x2, W_in, b_in, ln_in_g, ln_in_b)
    c0, c1, g0, g1 = _moe_layer(h, ln_g[0], ln_b[0], Wg[0], bg[0],
                                W1, b1, W2, b2, 0)
    h = _k_cmb(h, c0, c1, g0, g1)
    c0, c1, g0, g1 = _moe_layer(h, ln_g[1], ln_b[1], Wg[1], bg[1],
                                W1, b1, W2, b2, 1)
    out = _k_out(h, c0, c1, g0, g1, W_out, b_out, ln_out_g, ln_out_b)
    return out.reshape(1, S, DIN)
